# Initial kernel scaffold; baseline (speedup 1.0000x reference)
import jax
import jax.numpy as jnp
from jax.experimental import pallas as pl


def kernel(x, edge_index, W, b):
    raise NotImplementedError("write your pallas kernel here")


# speedup vs baseline: 7.7932x; 7.7932x over previous
import functools

import jax
import jax.numpy as jnp
from jax import lax
from jax.experimental import pallas as pl
from jax.experimental.pallas import tpu as pltpu
from jax.experimental.pallas import tpu_sc as plsc

N_NODES = 10000
HIDDEN = 128
K_HOPS = 10
ALPHA = 0.1

_NC = 2
_NS = 16
_NW = _NC * _NS


def _make_propagate(n_edges: int, batch: int = 80):
    assert n_edges % _NW == 0
    e_per_w = n_edges // _NW
    assert e_per_w % batch == 0 and batch % 8 == 0 and batch <= 128
    n_batches = e_per_w // batch
    rows_per_sub = (N_NODES // _NS) // 8 * 8
    rows_rem = N_NODES - _NS * rows_per_sub

    mesh = plsc.VectorSubcoreMesh(core_axis_name="c", subcore_axis_name="s",
                                  num_cores=_NC, num_subcores=_NS)

    @functools.partial(
        pl.kernel,
        out_type=jax.ShapeDtypeStruct((_NC, N_NODES, HIDDEN), jnp.float32),
        mesh=mesh,
        scratch_types=[
            pltpu.VMEM((batch,), jnp.int32),
            pltpu.VMEM((batch,), jnp.int32),
            pltpu.VMEM((batch, HIDDEN), jnp.float32),
            pltpu.VMEM_SHARED((N_NODES, HIDDEN), jnp.float32),
            pltpu.SemaphoreType.DMA,
        ],
    )
    def propagate(p_hbm, src_hbm, dst_hbm, zeros_hbm, out_hbm,
                  src_v, dst_v, rows_v, acc_sh, sem):
        cid = lax.axis_index("c")
        sid = lax.axis_index("s")
        wid = sid * _NC + cid
        base = wid * e_per_w

        r0 = pl.multiple_of(sid * rows_per_sub, 8)
        rr = _NS * rows_per_sub
        pltpu.sync_copy(zeros_hbm.at[pl.ds(r0, rows_per_sub)],
                        acc_sh.at[pl.ds(r0, rows_per_sub)])
        @pl.when(sid == 0)
        def _():
            pltpu.sync_copy(zeros_hbm.at[pl.ds(rr, rows_rem)],
                            acc_sh.at[pl.ds(rr, rows_rem)])
        plsc.subcore_barrier()

        def body(b, carry):
            off = pl.multiple_of(base + b * batch, 8)
            pltpu.sync_copy(src_hbm.at[pl.ds(off, batch)], src_v)
            pltpu.sync_copy(dst_hbm.at[pl.ds(off, batch)], dst_v)
            pltpu.async_copy(p_hbm.at[src_v], rows_v, sem).wait()
            pltpu.sync_copy(rows_v, acc_sh.at[dst_v], add=True)
            return carry

        lax.fori_loop(0, n_batches, body, 0, unroll=False)
        plsc.subcore_barrier()
        pltpu.sync_copy(acc_sh.at[pl.ds(r0, rows_per_sub)],
                        out_hbm.at[cid, pl.ds(r0, rows_per_sub)])
        @pl.when(sid == 0)
        def _():
            pltpu.sync_copy(acc_sh.at[pl.ds(rr, rows_rem)],
                            out_hbm.at[cid, pl.ds(rr, rows_rem)])

    return propagate


_BI = 1000


def _prep_body(x_ref, w_ref, b_ref, d2_ref, h0_ref, p0_ref, dinv_ref):
    h0 = lax.dot_general(x_ref[...], w_ref[...],
                         (((1,), (1,)), ((), ())),
                         preferred_element_type=jnp.float32,
                         precision=lax.Precision.HIGHEST)
    h0 = h0 + b_ref[...]
    deg = d2_ref[0, :, 0:1] + d2_ref[1, :, 0:1] + 1.0
    dinv = lax.rsqrt(deg)
    h0_ref[...] = h0
    p0_ref[...] = dinv * h0
    dinv_ref[...] = dinv


def _tc_prep(x, W, b, deg2):
    grid = (N_NODES // _BI,)
    return pl.pallas_call(
        _prep_body,
        grid=grid,
        in_specs=[
            pl.BlockSpec((_BI, HIDDEN), lambda i: (i, 0)),
            pl.BlockSpec((HIDDEN, HIDDEN), lambda i: (0, 0)),
            pl.BlockSpec((1, HIDDEN), lambda i: (0, 0)),
            pl.BlockSpec((_NC, _BI, HIDDEN), lambda i: (0, i, 0)),
        ],
        out_specs=[
            pl.BlockSpec((_BI, HIDDEN), lambda i: (i, 0)),
            pl.BlockSpec((_BI, HIDDEN), lambda i: (i, 0)),
            pl.BlockSpec((_BI, 1), lambda i: (i, 0)),
        ],
        out_shape=[
            jax.ShapeDtypeStruct((N_NODES, HIDDEN), jnp.float32),
            jax.ShapeDtypeStruct((N_NODES, HIDDEN), jnp.float32),
            jax.ShapeDtypeStruct((N_NODES, 1), jnp.float32),
        ],
    )(x, W, b.reshape(1, HIDDEN), deg2)


def _blend_body(is_last, a2_ref, p_ref, h0_ref, dinv_ref, out_ref):
    dinv = dinv_ref[...]
    agg = a2_ref[0] + a2_ref[1] + p_ref[...]
    h = (1.0 - ALPHA) * (dinv * agg) + ALPHA * h0_ref[...]
    out_ref[...] = h if is_last else dinv * h


def _tc_blend(agg2, p, h0, dinv, is_last):
    grid = (N_NODES // _BI,)
    return pl.pallas_call(
        functools.partial(_blend_body, is_last),
        grid=grid,
        in_specs=[
            pl.BlockSpec((_NC, _BI, HIDDEN), lambda i: (0, i, 0)),
            pl.BlockSpec((_BI, HIDDEN), lambda i: (i, 0)),
            pl.BlockSpec((_BI, HIDDEN), lambda i: (i, 0)),
            pl.BlockSpec((_BI, 1), lambda i: (i, 0)),
        ],
        out_specs=pl.BlockSpec((_BI, HIDDEN), lambda i: (i, 0)),
        out_shape=jax.ShapeDtypeStruct((N_NODES, HIDDEN), jnp.float32),
    )(agg2, p, h0, dinv)


def kernel(x, edge_index, W, b):
    src = edge_index[0].astype(jnp.int32)
    dst = edge_index[1].astype(jnp.int32)
    n_edges = src.shape[0]

    zeros = jnp.zeros((N_NODES, HIDDEN), jnp.float32)
    ones = jnp.ones((N_NODES, HIDDEN), jnp.float32)

    propagate = _make_propagate(n_edges)

    deg2 = propagate(ones, src, dst, zeros)
    h0, p, dinv = _tc_prep(x, W, b, deg2)

    for k in range(K_HOPS):
        agg2 = propagate(p, src, dst, zeros)
        p = _tc_blend(agg2, p, h0, dinv, is_last=(k == K_HOPS - 1))
    return p
